# baseline (speedup 1.0000x reference)
import functools
import math

import jax
import jax.numpy as jnp
from jax import lax
from jax.experimental import pallas as pl
from jax.experimental.pallas import tpu as pltpu
from jax.experimental.pallas import tpu_sc as plsc

TOP_K = 8
D_IN = 512
D_OUT = 1024
NUM_SLOTS = 8192
SCALE = 1.0 / math.sqrt(D_IN)

_NC = 2
_NS = 16
_NW = _NC * _NS
_L = 16


def _router_body(q_ref, kt_ref, rel_ref, relc_ref, w_ref, idx_ref):
    s = jax.lax.dot_general(
        q_ref[...], kt_ref[...], (((1,), (0,)), ((), ())),
        preferred_element_type=jnp.float32,
        precision=jax.lax.Precision.DEFAULT,
    )
    bias = jnp.log(rel_ref[...] + 1e-10)
    bias_col = jnp.log(relc_ref[...] + 1e-10)
    sb = s + bias
    bq = s.shape[0]
    iota = jax.lax.broadcasted_iota(jnp.int32, (bq, NUM_SLOTS), 1)
    neg = jnp.float32(-jnp.inf)
    raws = []
    idxs = []
    for _ in range(TOP_K):
        m = jnp.max(sb, axis=1, keepdims=True)
        masked = jnp.where(sb == m, iota, jnp.int32(2**30))
        idxk = jnp.min(masked, axis=1, keepdims=True)
        sel = masked == idxk
        sel_f = sel.astype(jnp.float32)
        bias_k = jax.lax.dot_general(
            sel_f, bias_col, (((1,), (0,)), ((), ())),
            preferred_element_type=jnp.float32,
            precision=jax.lax.Precision.HIGHEST,
        )
        raws.append(m - bias_k)
        idxs.append(idxk)
        sb = jnp.where(sel, neg, sb)
    raw = jnp.concatenate(raws, axis=1) * jnp.float32(SCALE)
    amax = jnp.max(raw, axis=1, keepdims=True)
    e = jnp.exp(raw - amax)
    w_ref[...] = e / jnp.sum(e, axis=1, keepdims=True)
    idx_ref[...] = jnp.concatenate(idxs, axis=1)


def _router_topk(q2, keys_t, rel2, relc):
    t_total = q2.shape[0]
    bq = 128
    grid = (t_total // bq,)
    return pl.pallas_call(
        _router_body,
        grid=grid,
        in_specs=[
            pl.BlockSpec((bq, D_IN), lambda i: (i, 0)),
            pl.BlockSpec((D_IN, NUM_SLOTS), lambda i: (0, 0)),
            pl.BlockSpec((1, NUM_SLOTS), lambda i: (0, 0)),
            pl.BlockSpec((NUM_SLOTS, 1), lambda i: (0, 0)),
        ],
        out_specs=[
            pl.BlockSpec((bq, TOP_K), lambda i: (i, 0)),
            pl.BlockSpec((bq, TOP_K), lambda i: (i, 0)),
        ],
        out_shape=[
            jax.ShapeDtypeStruct((t_total, TOP_K), jnp.float32),
            jax.ShapeDtypeStruct((t_total, TOP_K), jnp.int32),
        ],
    )(q2, keys_t, rel2, relc)


_G = 2


def _combine_body(values_hbm, idx_hbm, w_hbm, out_hbm,
                  idx_v, w_v, rows0, rows1, out_v, sem0, sem1):
    t_per_w = idx_v.shape[0] // TOP_K
    n_groups = t_per_w // _G
    wid = lax.axis_index("c") * _NS + lax.axis_index("s")
    base = wid * t_per_w
    pltpu.sync_copy(idx_hbm.at[pl.ds(base * TOP_K, t_per_w * TOP_K)], idx_v)
    pltpu.sync_copy(w_hbm.at[pl.ds(base * _L, t_per_w * _L)], w_v)

    def gather(g, rows, sem):
        return pltpu.async_copy(
            values_hbm.at[idx_v.at[pl.ds(g * (_G * TOP_K), _G * TOP_K)]],
            rows, sem)

    def wait(g, rows, sem):
        pltpu.make_async_copy(
            values_hbm.at[idx_v.at[pl.ds(g * (_G * TOP_K), _G * TOP_K)]],
            rows, sem).wait()

    def compute(g, rows):
        for s in range(_G):
            wrow = w_v[pl.ds((g * _G + s) * _L, _L)]
            wvs = [jnp.full((_L,), wrow[j], jnp.float32)
                   for j in range(TOP_K)]
            for c in range(D_OUT // _L):
                a0 = wvs[0] * rows[s * TOP_K + 0, pl.ds(c * _L, _L)]
                a1 = wvs[1] * rows[s * TOP_K + 1, pl.ds(c * _L, _L)]
                for j in range(2, TOP_K, 2):
                    a0 = a0 + wvs[j] * rows[s * TOP_K + j, pl.ds(c * _L, _L)]
                    a1 = a1 + wvs[j + 1] * rows[s * TOP_K + j + 1,
                                                pl.ds(c * _L, _L)]
                out_v[s, pl.ds(c * _L, _L)] = a0 + a1
        pltpu.sync_copy(out_v, out_hbm.at[pl.ds(base + g * _G, _G)])

    gather(0, rows0, sem0)

    def body(i, carry):
        g0 = 2 * i
        g1 = 2 * i + 1
        gather(g1, rows1, sem1)
        wait(g0, rows0, sem0)
        compute(g0, rows0)

        @pl.when(i < n_groups // 2 - 1)
        def _():
            gather(g0 + 2, rows0, sem0)

        wait(g1, rows1, sem1)
        compute(g1, rows1)
        return carry

    lax.fori_loop(0, n_groups // 2, body, 0)


def _combine(values, idx, w):
    t_total = idx.shape[0]
    t_per_w = t_total // _NW
    mesh = plsc.VectorSubcoreMesh(core_axis_name="c", subcore_axis_name="s")
    f = functools.partial(
        pl.kernel,
        out_type=jax.ShapeDtypeStruct((t_total, D_OUT), jnp.float32),
        mesh=mesh,
        scratch_types=[
            pltpu.VMEM((t_per_w * TOP_K,), jnp.int32),
            pltpu.VMEM((t_per_w * _L,), jnp.float32),
            pltpu.VMEM((_G * TOP_K, D_OUT), jnp.float32),
            pltpu.VMEM((_G * TOP_K, D_OUT), jnp.float32),
            pltpu.VMEM((_G, D_OUT), jnp.float32),
            pltpu.SemaphoreType.DMA,
            pltpu.SemaphoreType.DMA,
        ],
    )(_combine_body)
    w16 = jnp.pad(w, ((0, 0), (0, _L - TOP_K))).reshape(t_total * _L)
    return f(values, idx.reshape(t_total * TOP_K), w16)


def kernel(query, keys, values, reliability):
    b, s, _ = query.shape
    t_total = b * s
    q2 = query.reshape(t_total, D_IN)
    keys_t = keys.T
    rel2 = reliability.reshape(1, NUM_SLOTS)
    relc = reliability.reshape(NUM_SLOTS, 1)
    n_chunks = 4
    tc = t_total // n_chunks
    ws, outs = [], []
    for p in range(n_chunks):
        wp, ip = _router_topk(q2[p * tc:(p + 1) * tc], keys_t, rel2, relc)
        ws.append(wp)
        outs.append(_combine(values, ip, wp))
    out = jnp.concatenate(outs, axis=0)
    w = jnp.concatenate(ws, axis=0)
    return (out.reshape(b, s, D_OUT), w.reshape(b, s, TOP_K))


# speedup vs baseline: 3.2056x; 3.2056x over previous
import functools
import math

import jax
import jax.numpy as jnp
from jax import lax
from jax.experimental import pallas as pl
from jax.experimental.pallas import tpu as pltpu
from jax.experimental.pallas import tpu_sc as plsc

TOP_K = 8
D_IN = 512
D_OUT = 1024
NUM_SLOTS = 8192
SCALE = 1.0 / math.sqrt(D_IN)

_NC = 2
_NS = 16
_NW = _NC * _NS
_L = 16


def _router_body(q_ref, kt_ref, rel_ref, w_ref, idx_ref):
    s = jax.lax.dot_general(
        q_ref[...], kt_ref[...], (((1,), (0,)), ((), ())),
        preferred_element_type=jnp.float32,
        precision=jax.lax.Precision.DEFAULT,
    )
    bias = jnp.log(rel_ref[...] + 1e-10)
    sb = s + bias
    bq = s.shape[0]
    iota = jax.lax.broadcasted_iota(jnp.int32, (bq, NUM_SLOTS), 1)
    bias_b = jnp.broadcast_to(bias, (bq, NUM_SLOTS))
    neg = jnp.float32(-jnp.inf)
    raws = []
    idxs = []
    for _ in range(TOP_K):
        m = jnp.max(sb, axis=1, keepdims=True)
        masked = jnp.where(sb == m, iota, jnp.int32(2**30))
        idxk = jnp.min(masked, axis=1, keepdims=True)
        sel = masked == idxk
        bias_k = jnp.sum(jnp.where(sel, bias_b, 0.0), axis=1, keepdims=True)
        raws.append(m - bias_k)
        idxs.append(idxk)
        sb = jnp.where(sel, neg, sb)
    raw = jnp.concatenate(raws, axis=1) * jnp.float32(SCALE)
    amax = jnp.max(raw, axis=1, keepdims=True)
    e = jnp.exp(raw - amax)
    w_ref[...] = e / jnp.sum(e, axis=1, keepdims=True)
    idx_ref[...] = jnp.concatenate(idxs, axis=1)


def _router_topk(q2, keys_t, rel2):
    t_total = q2.shape[0]
    bq = 128
    grid = (t_total // bq,)
    return pl.pallas_call(
        _router_body,
        grid=grid,
        in_specs=[
            pl.BlockSpec((bq, D_IN), lambda i: (i, 0)),
            pl.BlockSpec((D_IN, NUM_SLOTS), lambda i: (0, 0)),
            pl.BlockSpec((1, NUM_SLOTS), lambda i: (0, 0)),
        ],
        out_specs=[
            pl.BlockSpec((bq, TOP_K), lambda i: (i, 0)),
            pl.BlockSpec((bq, TOP_K), lambda i: (i, 0)),
        ],
        out_shape=[
            jax.ShapeDtypeStruct((t_total, TOP_K), jnp.float32),
            jax.ShapeDtypeStruct((t_total, TOP_K), jnp.int32),
        ],
    )(q2, keys_t, rel2)


_G = 2


def _combine_body(values_hbm, idx_hbm, w_hbm, out_hbm,
                  idx_v, w_v, rows0, rows1, out_v, sem0, sem1):
    t_per_w = idx_v.shape[0] // TOP_K
    n_groups = t_per_w // _G
    wid = lax.axis_index("c") * _NS + lax.axis_index("s")
    base = wid * t_per_w
    pltpu.sync_copy(idx_hbm.at[pl.ds(base * TOP_K, t_per_w * TOP_K)], idx_v)
    pltpu.sync_copy(w_hbm.at[pl.ds(base * _L, t_per_w * _L)], w_v)

    def gather(g, rows, sem):
        return pltpu.async_copy(
            values_hbm.at[idx_v.at[pl.ds(g * (_G * TOP_K), _G * TOP_K)]],
            rows, sem)

    def wait(g, rows, sem):
        pltpu.make_async_copy(
            values_hbm.at[idx_v.at[pl.ds(g * (_G * TOP_K), _G * TOP_K)]],
            rows, sem).wait()

    def compute(g, rows):
        for s in range(_G):
            wrow = w_v[pl.ds((g * _G + s) * _L, _L)]
            wvs = [jnp.full((_L,), wrow[j], jnp.float32)
                   for j in range(TOP_K)]
            for c in range(D_OUT // _L):
                a0 = wvs[0] * rows[s * TOP_K + 0, pl.ds(c * _L, _L)]
                a1 = wvs[1] * rows[s * TOP_K + 1, pl.ds(c * _L, _L)]
                for j in range(2, TOP_K, 2):
                    a0 = a0 + wvs[j] * rows[s * TOP_K + j, pl.ds(c * _L, _L)]
                    a1 = a1 + wvs[j + 1] * rows[s * TOP_K + j + 1,
                                                pl.ds(c * _L, _L)]
                out_v[s, pl.ds(c * _L, _L)] = a0 + a1
        pltpu.sync_copy(out_v, out_hbm.at[pl.ds(base + g * _G, _G)])

    gather(0, rows0, sem0)

    def body(i, carry):
        g0 = 2 * i
        g1 = 2 * i + 1
        gather(g1, rows1, sem1)
        wait(g0, rows0, sem0)
        compute(g0, rows0)

        @pl.when(i < n_groups // 2 - 1)
        def _():
            gather(g0 + 2, rows0, sem0)

        wait(g1, rows1, sem1)
        compute(g1, rows1)
        return carry

    lax.fori_loop(0, n_groups // 2, body, 0)


def _combine(values, idx, w):
    t_total = idx.shape[0]
    t_per_w = t_total // _NW
    mesh = plsc.VectorSubcoreMesh(core_axis_name="c", subcore_axis_name="s")
    f = functools.partial(
        pl.kernel,
        out_type=jax.ShapeDtypeStruct((t_total, D_OUT), jnp.float32),
        mesh=mesh,
        scratch_types=[
            pltpu.VMEM((t_per_w * TOP_K,), jnp.int32),
            pltpu.VMEM((t_per_w * _L,), jnp.float32),
            pltpu.VMEM((_G * TOP_K, D_OUT), jnp.float32),
            pltpu.VMEM((_G * TOP_K, D_OUT), jnp.float32),
            pltpu.VMEM((_G, D_OUT), jnp.float32),
            pltpu.SemaphoreType.DMA,
            pltpu.SemaphoreType.DMA,
        ],
    )(_combine_body)
    w16 = jnp.pad(w, ((0, 0), (0, _L - TOP_K))).reshape(t_total * _L)
    return f(values, idx.reshape(t_total * TOP_K), w16)


def kernel(query, keys, values, reliability):
    b, s, _ = query.shape
    t_total = b * s
    q2 = query.reshape(t_total, D_IN)
    keys_t = keys.T
    rel2 = reliability.reshape(1, NUM_SLOTS)
    n_chunks = 4
    tc = t_total // n_chunks
    ws, outs = [], []
    for p in range(n_chunks):
        wp, ip = _router_topk(q2[p * tc:(p + 1) * tc], keys_t, rel2)
        ws.append(wp)
        outs.append(_combine(values, ip, wp))
    out = jnp.concatenate(outs, axis=0)
    w = jnp.concatenate(ws, axis=0)
    return (out.reshape(b, s, D_OUT), w.reshape(b, s, TOP_K))
